# baseline (speedup 1.0000x reference)
import functools

import jax
import jax.numpy as jnp
from jax import lax
from jax.experimental import pallas as pl
from jax.experimental.pallas import tpu as pltpu
from jax.experimental.pallas import tpu_sc as plsc

NUM_CORES = 2
NUM_SUBCORES = 16
LANES = 16
NW = NUM_CORES * NUM_SUBCORES

B = 16384
L = 200
DIM = 16
VOCAB_SIZE = 1000000

EPW = B // NW
G = 16
GROUPS = EPW // G
ROWS_G = G * L
CHUNK = 3200
NCHUNK = ROWS_G // CHUNK

EW_BLK = 4096

_MAGIC = 12582912.0


def _ew_body(emb_ref, w_ref, out_ref):
    out_ref[...] = jnp.sum(emb_ref[...] * w_ref[...], axis=0)


_tc_ew = pl.pallas_call(
    _ew_body,
    out_shape=jax.ShapeDtypeStruct((VOCAB_SIZE,), jnp.float32),
    grid=(pl.cdiv(VOCAB_SIZE, EW_BLK),),
    in_specs=[
        pl.BlockSpec((DIM, EW_BLK), lambda i: (0, i)),
        pl.BlockSpec((DIM, 1), lambda i: (0, 0)),
    ],
    out_specs=pl.BlockSpec((EW_BLK,), lambda i: (i,)),
)


def _sc_body(xt_hbm, ew_hbm, b_hbm, out_hbm,
             slab_v, idx0, idx1, val0, val1, b_v, out_v, ew_sh,
             gsem0, gsem1, ssem):
    c = lax.axis_index("c")
    s = lax.axis_index("s")
    wid = c * NUM_SUBCORES + s
    base = wid * EPW

    @pl.when(s == 0)
    def _():
        pltpu.sync_copy(ew_hbm, ew_sh)
    pltpu.sync_copy(b_hbm, b_v)
    bvec = b_v[...]
    plsc.subcore_barrier()

    def slab_start(g):
        pltpu.async_copy(xt_hbm.at[:, pl.ds(base + g * G, G)], slab_v, ssem)

    def slab_wait():
        pltpu.make_async_copy(xt_hbm.at[:, pl.ds(0, G)], slab_v, ssem).wait()

    def repack(idx_v):
        def body(r, carry):
            rb = r * 8
            for k in range(8):
                idx_v[pl.ds((rb + k) * LANES, LANES)] = slab_v[rb + k]
            return carry
        lax.fori_loop(0, L // 8, body, 0)

    HALF = ROWS_G // 2

    def fire(idx_v, val_v, gsem):
        pltpu.async_copy(
            ew_sh.at[idx_v.at[pl.ds(0, HALF)]],
            val_v.at[pl.ds(0, HALF)], gsem)
        pltpu.async_copy(
            ew_sh.at[idx_v.at[pl.ds(HALF, HALF)]],
            val_v.at[pl.ds(HALF, HALF)], gsem)

    def drain(idx_v, val_v, gsem):
        pltpu.make_async_copy(
            ew_sh.at[idx_v.at[pl.ds(0, HALF)]],
            val_v.at[pl.ds(0, HALF)], gsem).wait()
        pltpu.make_async_copy(
            ew_sh.at[idx_v.at[pl.ds(HALF, HALF)]],
            val_v.at[pl.ds(HALF, HALF)], gsem).wait()

    def compute(g, val_v):
        def body(i, accs):
            a0, a1, a2, a3 = accs
            rb = i * (8 * LANES)
            a0 = a0 + val_v[pl.ds(rb, LANES)]
            a1 = a1 + val_v[pl.ds(rb + LANES, LANES)]
            a2 = a2 + val_v[pl.ds(rb + 2 * LANES, LANES)]
            a3 = a3 + val_v[pl.ds(rb + 3 * LANES, LANES)]
            a0 = a0 + val_v[pl.ds(rb + 4 * LANES, LANES)]
            a1 = a1 + val_v[pl.ds(rb + 5 * LANES, LANES)]
            a2 = a2 + val_v[pl.ds(rb + 6 * LANES, LANES)]
            a3 = a3 + val_v[pl.ds(rb + 7 * LANES, LANES)]
            return (a0, a1, a2, a3)

        z16 = jnp.zeros((LANES,), jnp.float32)
        a0, a1, a2, a3 = lax.fori_loop(0, L // 8, body,
                                       (z16, z16, z16, z16))
        z = ((a0 + a1) + (a2 + a3)) * (1.0 / L) + bvec
        p = 1.0 / (1.0 + jnp.exp(-z))
        r = p * 10000.0
        r = (r + _MAGIC) - _MAGIC
        out_v[pl.ds(g * G, G)] = r / 10000.0

    pltpu.sync_copy(xt_hbm.at[:, pl.ds(base, G)], slab_v)
    repack(idx0)
    fire(idx0, val0, gsem0)
    slab_start(1)

    def pipe_body(i, carry):
        a = 2 * i
        slab_wait()
        repack(idx1)
        fire(idx1, val1, gsem1)
        slab_start(a + 2)
        drain(idx0, val0, gsem0)
        compute(a, val0)
        slab_wait()
        repack(idx0)
        fire(idx0, val0, gsem0)
        slab_start(a + 3)
        drain(idx1, val1, gsem1)
        compute(a + 1, val1)
        return carry

    lax.fori_loop(0, GROUPS // 2 - 1, pipe_body, 0)

    slab_wait()
    repack(idx1)
    fire(idx1, val1, gsem1)
    drain(idx0, val0, gsem0)
    compute(GROUPS - 2, val0)
    drain(idx1, val1, gsem1)
    compute(GROUPS - 1, val1)

    pltpu.sync_copy(out_v, out_hbm.at[pl.ds(base, EPW)])


@functools.partial(
    pl.kernel,
    out_type=jax.ShapeDtypeStruct((B,), jnp.float32),
    mesh=plsc.VectorSubcoreMesh(core_axis_name="c", subcore_axis_name="s",
                                num_cores=NUM_CORES,
                                num_subcores=NUM_SUBCORES),
    scratch_types=[
        pltpu.VMEM((L, G), jnp.int32),
        pltpu.VMEM((ROWS_G,), jnp.int32),
        pltpu.VMEM((ROWS_G,), jnp.int32),
        pltpu.VMEM((ROWS_G,), jnp.float32),
        pltpu.VMEM((ROWS_G,), jnp.float32),
        pltpu.VMEM((LANES,), jnp.float32),
        pltpu.VMEM((EPW,), jnp.float32),
        pltpu.VMEM_SHARED((VOCAB_SIZE,), jnp.float32),
        pltpu.SemaphoreType.DMA,
        pltpu.SemaphoreType.DMA,
        pltpu.SemaphoreType.DMA,
    ],
    compiler_params=pltpu.CompilerParams(use_tc_tiling_on_sc=False),
)
def _sc_kernel(xt_hbm, ew_hbm, b_hbm, out_hbm,
               slab_v, idx0, idx1, val0, val1, b_v, out_v, ew_sh,
               gsem0, gsem1, ssem):
    _sc_body(xt_hbm, ew_hbm, b_hbm, out_hbm,
             slab_v, idx0, idx1, val0, val1, b_v, out_v, ew_sh,
             gsem0, gsem1, ssem)


@jax.jit
def kernel(x, embed, W, b):
    emb_t = embed.T
    ew = _tc_ew(emb_t, W.astype(jnp.float32))
    xt = x.T.astype(jnp.int32)
    b16 = jnp.broadcast_to(b, (LANES,)).astype(jnp.float32)
    y = _sc_kernel(xt, ew, b16)
    return y.reshape(B, 1)


# speedup vs baseline: 1.0009x; 1.0009x over previous
import functools

import jax
import jax.numpy as jnp
from jax import lax
from jax.experimental import pallas as pl
from jax.experimental.pallas import tpu as pltpu
from jax.experimental.pallas import tpu_sc as plsc

NUM_CORES = 2
NUM_SUBCORES = 16
LANES = 16
NW = NUM_CORES * NUM_SUBCORES

B = 16384
L = 200
DIM = 16
VOCAB_SIZE = 1000000

EPW = B // NW
G = 16
GROUPS = EPW // G
ROWS_G = G * L
CHUNK = 3200
NCHUNK = ROWS_G // CHUNK

EW_BLK = 4096

_MAGIC = 12582912.0


def _ew_body(emb_ref, w_ref, out_ref):
    out_ref[...] = jnp.sum(emb_ref[...] * w_ref[...], axis=0)


_tc_ew = pl.pallas_call(
    _ew_body,
    out_shape=jax.ShapeDtypeStruct((VOCAB_SIZE,), jnp.float32),
    grid=(pl.cdiv(VOCAB_SIZE, EW_BLK),),
    in_specs=[
        pl.BlockSpec((DIM, EW_BLK), lambda i: (0, i)),
        pl.BlockSpec((DIM, 1), lambda i: (0, 0)),
    ],
    out_specs=pl.BlockSpec((EW_BLK,), lambda i: (i,)),
)


def _sc_body(xt_hbm, ew_hbm, b_hbm, out_hbm,
             slab_v, idx0, idx1, val0, val1, b_v, out_v, ew_sh,
             gsem0, gsem1, ssem):
    c = lax.axis_index("c")
    s = lax.axis_index("s")
    wid = c * NUM_SUBCORES + s
    base = wid * EPW

    @pl.when(s == 0)
    def _():
        pltpu.sync_copy(ew_hbm, ew_sh)
    pltpu.sync_copy(b_hbm, b_v)
    bvec = b_v[...]
    plsc.subcore_barrier()

    def slab_start(g):
        pltpu.async_copy(xt_hbm.at[:, pl.ds(base + g * G, G)], slab_v, ssem)

    def slab_wait():
        pltpu.make_async_copy(xt_hbm.at[:, pl.ds(0, G)], slab_v, ssem).wait()

    def repack(idx_v):
        def body(r, carry):
            rb = r * 8
            for k in range(8):
                idx_v[pl.ds((rb + k) * LANES, LANES)] = slab_v[rb + k]
            return carry
        lax.fori_loop(0, L // 8, body, 0)

    def fire(idx_v, val_v, gsem):
        for k in range(NCHUNK):
            pltpu.async_copy(
                ew_sh.at[idx_v.at[pl.ds(k * CHUNK, CHUNK)]],
                val_v.at[pl.ds(k * CHUNK, CHUNK)], gsem)

    def drain(idx_v, val_v, gsem):
        for k in range(NCHUNK):
            pltpu.make_async_copy(
                ew_sh.at[idx_v.at[pl.ds(k * CHUNK, CHUNK)]],
                val_v.at[pl.ds(k * CHUNK, CHUNK)], gsem).wait()

    def compute(g, val_v):
        def body(i, accs):
            a0, a1, a2, a3 = accs
            rb = i * (8 * LANES)
            a0 = a0 + val_v[pl.ds(rb, LANES)]
            a1 = a1 + val_v[pl.ds(rb + LANES, LANES)]
            a2 = a2 + val_v[pl.ds(rb + 2 * LANES, LANES)]
            a3 = a3 + val_v[pl.ds(rb + 3 * LANES, LANES)]
            a0 = a0 + val_v[pl.ds(rb + 4 * LANES, LANES)]
            a1 = a1 + val_v[pl.ds(rb + 5 * LANES, LANES)]
            a2 = a2 + val_v[pl.ds(rb + 6 * LANES, LANES)]
            a3 = a3 + val_v[pl.ds(rb + 7 * LANES, LANES)]
            return (a0, a1, a2, a3)

        z16 = jnp.zeros((LANES,), jnp.float32)
        a0, a1, a2, a3 = lax.fori_loop(0, L // 8, body,
                                       (z16, z16, z16, z16))
        z = ((a0 + a1) + (a2 + a3)) * (1.0 / L) + bvec
        p = 1.0 / (1.0 + jnp.exp(-z))
        r = p * 10000.0
        r = (r + _MAGIC) - _MAGIC
        out_v[pl.ds(g * G, G)] = r / 10000.0

    pltpu.sync_copy(xt_hbm.at[:, pl.ds(base, G)], slab_v)
    repack(idx0)
    fire(idx0, val0, gsem0)
    slab_start(1)

    def pipe_body(i, carry):
        a = 2 * i
        slab_wait()
        repack(idx1)
        fire(idx1, val1, gsem1)
        slab_start(a + 2)
        drain(idx0, val0, gsem0)
        compute(a, val0)
        slab_wait()
        repack(idx0)
        fire(idx0, val0, gsem0)
        slab_start(a + 3)
        drain(idx1, val1, gsem1)
        compute(a + 1, val1)
        return carry

    lax.fori_loop(0, GROUPS // 2 - 1, pipe_body, 0)

    slab_wait()
    repack(idx1)
    fire(idx1, val1, gsem1)
    drain(idx0, val0, gsem0)
    compute(GROUPS - 2, val0)
    drain(idx1, val1, gsem1)
    compute(GROUPS - 1, val1)

    pltpu.sync_copy(out_v, out_hbm.at[pl.ds(base, EPW)])


@functools.partial(
    pl.kernel,
    out_type=jax.ShapeDtypeStruct((B,), jnp.float32),
    mesh=plsc.VectorSubcoreMesh(core_axis_name="c", subcore_axis_name="s",
                                num_cores=NUM_CORES,
                                num_subcores=NUM_SUBCORES),
    scratch_types=[
        pltpu.VMEM((L, G), jnp.int32),
        pltpu.VMEM((ROWS_G,), jnp.int32),
        pltpu.VMEM((ROWS_G,), jnp.int32),
        pltpu.VMEM((ROWS_G,), jnp.float32),
        pltpu.VMEM((ROWS_G,), jnp.float32),
        pltpu.VMEM((LANES,), jnp.float32),
        pltpu.VMEM((EPW,), jnp.float32),
        pltpu.VMEM_SHARED((VOCAB_SIZE,), jnp.float32),
        pltpu.SemaphoreType.DMA,
        pltpu.SemaphoreType.DMA,
        pltpu.SemaphoreType.DMA,
    ],
    compiler_params=pltpu.CompilerParams(use_tc_tiling_on_sc=False),
)
def _sc_kernel(xt_hbm, ew_hbm, b_hbm, out_hbm,
               slab_v, idx0, idx1, val0, val1, b_v, out_v, ew_sh,
               gsem0, gsem1, ssem):
    _sc_body(xt_hbm, ew_hbm, b_hbm, out_hbm,
             slab_v, idx0, idx1, val0, val1, b_v, out_v, ew_sh,
             gsem0, gsem1, ssem)


@jax.jit
def kernel(x, embed, W, b):
    emb_t = embed.T
    ew = _tc_ew(emb_t, W.astype(jnp.float32))
    xt = x.T.astype(jnp.int32)
    b16 = jnp.broadcast_to(b, (LANES,)).astype(jnp.float32)
    y = _sc_kernel(xt, ew, b16)
    return y.reshape(B, 1)
